# Initial kernel scaffold; baseline (speedup 1.0000x reference)
import jax
import jax.numpy as jnp
from jax.experimental import pallas as pl


def kernel(txt_input, syl_input, W_char, W_syl):
    raise NotImplementedError("write your pallas kernel here")


# speedup vs baseline: 4.2201x; 4.2201x over previous
import functools

import jax
import jax.numpy as jnp
from jax import lax
from jax.experimental import pallas as pl
from jax.experimental.pallas import tpu as pltpu
from jax.experimental.pallas import tpu_sc as plsc

EMBED = 64
CHUNK = 128
NBUF = 4


@functools.lru_cache(maxsize=None)
def _build(nw, nc, per_w):
    nchunks = per_w // CHUNK
    ngroups = nchunks // NBUF
    n2 = nw * per_w
    mesh = plsc.VectorSubcoreMesh(core_axis_name="c", subcore_axis_name="s")

    @functools.partial(
        pl.kernel,
        mesh=mesh,
        compiler_params=pltpu.CompilerParams(use_tc_tiling_on_sc=False),
        out_type=jax.ShapeDtypeStruct((n2, EMBED), jnp.float32),
        scratch_types=[
            pltpu.VMEM((nchunks, CHUNK), jnp.int32),
            pltpu.VMEM((NBUF, CHUNK, EMBED), jnp.float32),
            pltpu.SemaphoreType.DMA,
            pltpu.SemaphoreType.DMA,
        ],
    )
    def emb(table, idx, out, idx_v, buf, gsem, wsem):
        wid = lax.axis_index("s") * nc + lax.axis_index("c")
        base = wid * per_w
        pltpu.sync_copy(idx.at[wid], idx_v)

        def group(g, carry):
            gcps = []
            for b in range(NBUF):
                j = g * NBUF + b
                gcps.append(
                    pltpu.async_copy(table.at[idx_v.at[j]], buf.at[b], gsem))
            wcps = []
            for b in range(NBUF):
                j = g * NBUF + b
                gcps[b].wait()
                row = base + j * CHUNK
                wcps.append(pltpu.async_copy(
                    buf.at[b], out.at[pl.ds(row, CHUNK)], wsem))
            for w in wcps:
                w.wait()
            return carry

        lax.fori_loop(0, ngroups, group, 0)

    return emb


def kernel(txt_input, syl_input, W_char, W_syl):
    b, s = txt_input.shape
    n = b * s
    v = W_char.shape[0]
    info = plsc.get_sparse_core_info()
    nc, ns = info.num_cores, info.num_subcores
    nw = nc * ns
    per_w = (2 * n) // nw
    table = jnp.concatenate([W_char, W_syl], axis=0)
    idx = jnp.stack(
        [txt_input.astype(jnp.int32).reshape(n),
         syl_input.astype(jnp.int32).reshape(n) + v],
        axis=-1,
    ).reshape(nw, per_w // CHUNK, CHUNK)
    emb = _build(nw, nc, per_w)
    out = emb(table, idx)
    return out.reshape(b, s, 2 * EMBED)


# speedup vs baseline: 15.5660x; 3.6885x over previous
import functools

import jax
import jax.numpy as jnp
from jax import lax
from jax.experimental import pallas as pl
from jax.experimental.pallas import tpu as pltpu
from jax.experimental.pallas import tpu_sc as plsc

EMBED = 64
CHUNK = 128
NBUF = 4


@functools.lru_cache(maxsize=None)
def _build(nw, nc, per_w):
    nchunks = per_w // CHUNK
    ngroups = nchunks // NBUF
    n = nw * per_w
    mesh = plsc.VectorSubcoreMesh(core_axis_name="c", subcore_axis_name="s")

    @functools.partial(
        pl.kernel,
        mesh=mesh,
        compiler_params=pltpu.CompilerParams(use_tc_tiling_on_sc=False),
        out_type=jax.ShapeDtypeStruct((n, 2 * EMBED), jnp.float32),
        scratch_types=[
            pltpu.VMEM((nchunks, CHUNK), jnp.int32),
            pltpu.VMEM((nchunks, CHUNK), jnp.int32),
            pltpu.VMEM((NBUF, CHUNK, EMBED), jnp.float32),
            pltpu.VMEM((NBUF, CHUNK, EMBED), jnp.float32),
            pltpu.SemaphoreType.DMA,
            pltpu.SemaphoreType.DMA,
            pltpu.SemaphoreType.DMA,
        ],
    )
    def emb(w_char, w_syl, idx_c, idx_s, out, idxc_v, idxs_v, bufc, bufs,
            gsem, ssem, wsem):
        wid = lax.axis_index("s") * nc + lax.axis_index("c")
        base = wid * per_w
        pltpu.sync_copy(idx_c.at[wid], idxc_v)
        pltpu.sync_copy(idx_s.at[wid], idxs_v)

        def group(g, carry):
            gcps = []
            for b in range(NBUF):
                j = g * NBUF + b
                gcps.append((
                    pltpu.async_copy(w_char.at[idxc_v.at[j]], bufc.at[b], gsem),
                    pltpu.async_copy(w_syl.at[idxs_v.at[j]], bufs.at[b], ssem),
                ))
            wcps = []
            for b in range(NBUF):
                j = g * NBUF + b
                cc, cs = gcps[b]
                cc.wait()
                cs.wait()
                row = base + j * CHUNK
                wcps.append(pltpu.async_copy(
                    bufc.at[b], out.at[pl.ds(row, CHUNK), pl.ds(0, EMBED)],
                    wsem))
                wcps.append(pltpu.async_copy(
                    bufs.at[b], out.at[pl.ds(row, CHUNK), pl.ds(EMBED, EMBED)],
                    wsem))
            for w in wcps:
                w.wait()
            return carry

        lax.fori_loop(0, ngroups, group, 0)

    return emb


def kernel(txt_input, syl_input, W_char, W_syl):
    b, s = txt_input.shape
    n = b * s
    info = plsc.get_sparse_core_info()
    nc, ns = info.num_cores, info.num_subcores
    nw = nc * ns
    per_w = n // nw
    idx_c = txt_input.astype(jnp.int32).reshape(nw, per_w // CHUNK, CHUNK)
    idx_s = syl_input.astype(jnp.int32).reshape(nw, per_w // CHUNK, CHUNK)
    emb = _build(nw, nc, per_w)
    out = emb(W_char, W_syl, idx_c, idx_s)
    return out.reshape(b, s, 2 * EMBED)
